# baseline (speedup 1.0000x reference)
import functools

import jax
import jax.numpy as jnp
from jax import lax
from jax.experimental import pallas as pl
from jax.experimental.pallas import tpu as pltpu
from jax.experimental.pallas import tpu_sc as plsc

_B = 16384
_D = 64
_NC = 2
_NS = 16
_NW = _NC * _NS
_BPW = _B // _NW
_G = 16
_NG = _BPW // _G
_NB = 2


def _body(ph, pr, pt, nh, nr, nt, ent3, rel2, pos_out, neg_out,
          rh, rr, rt, dr, out_v, *setargs):
    wid = lax.axis_index("s") * _NC + lax.axis_index("c")
    sets = [(setargs[3 * b], setargs[3 * b + 1], setargs[3 * b + 2])
            for b in range(_NB)]

    def fire(g, b):
        bht, brl, sem = sets[b]
        v_h = rh[pl.ds(g * _G, _G)]
        v_t = rt[pl.ds(g * _G, _G)]
        for i in range(_G):
            pltpu.async_copy(ent3.at[lax.shift_right_logical(v_h[i], 3)],
                             bht.at[i], sem)
            pltpu.async_copy(ent3.at[lax.shift_right_logical(v_t[i], 3)],
                             bht.at[i + _G], sem)
        pltpu.async_copy(rel2.at[dr.at[pl.ds(g * _G, _G)]], brl, sem)

    def drain(b):
        bht, brl, sem = sets[b]
        pltpu.make_async_copy(ent3.at[pl.ds(0, 2 * _G)], bht, sem).wait()
        pltpu.make_async_copy(rel2.at[pl.ds(0, _G)], brl, sem).wait()

    def compute(g, b):
        bht, brl, _ = sets[b]
        v_h = rh[pl.ds(g * _G, _G)]
        v_t = rt[pl.ds(g * _G, _G)]
        rows = lax.iota(jnp.int32, 16)
        rows_t = rows + _G
        k_h = v_h & 7
        k_t = v_t & 7
        base_r = (rr[pl.ds(g * _G, _G)] & 1) * _D
        acc = jnp.zeros((16,), jnp.float32)
        for j in range(_D):
            jv = lax.broadcast(j, (16,))
            h = plsc.load_gather(bht, [rows, k_h, jv])
            t = plsc.load_gather(bht, [rows_t, k_t, jv])
            r = plsc.load_gather(brl, [rows, base_r + j])
            d = h + r - t
            acc = acc + d * d
        out_v[pl.ds(g * _G, 16)] = acc

    def do_term(hi, ri, ti, out_hbm):
        pltpu.sync_copy(hi.at[wid], rh)
        pltpu.sync_copy(ri.at[wid], rr)
        pltpu.sync_copy(ti.at[wid], rt)
        for s in range(_BPW // 16):
            v = rr[pl.ds(s * 16, 16)]
            dr[pl.ds(s * 16, 16)] = lax.shift_right_logical(v, 1)
        for b in range(_NB):
            fire(b, b)

        def qbody(q, _):
            for b in range(_NB):
                g = q * _NB + b
                drain(b)
                fire(g + _NB, b)
                compute(g, b)
            return 0

        lax.fori_loop(0, (_NG - _NB) // _NB, qbody, 0)
        for g in range(_NG - _NB, _NG):
            b = g % _NB
            drain(b)
            compute(g, b)
        pltpu.sync_copy(out_v, out_hbm.at[pl.ds(wid * _BPW, _BPW)])

    do_term(ph, pr, pt, pos_out)
    do_term(nh, nr, nt, neg_out)


@functools.partial(jax.jit)
def kernel(ph, pr, pt, nh, nr, nt, ent_embed, rel_embed):
    idxs = [x.astype(jnp.int32).reshape(_NW, _BPW)
            for x in (ph, pr, pt, nh, nr, nt)]
    ent3 = ent_embed.reshape(ent_embed.shape[0] // 8, 8, _D)
    rel2 = rel_embed.reshape(rel_embed.shape[0] // 2, 2 * _D)
    mesh = plsc.VectorSubcoreMesh(core_axis_name="c", subcore_axis_name="s",
                                  num_cores=_NC, num_subcores=_NS)
    set_scratch = []
    for _ in range(_NB):
        set_scratch += [
            pltpu.VMEM((2 * _G, 8, _D), jnp.float32),
            pltpu.VMEM((_G, 2 * _D), jnp.float32),
            pltpu.SemaphoreType.DMA,
        ]
    f = pl.kernel(
        _body,
        out_type=(jax.ShapeDtypeStruct((_B,), jnp.float32),
                  jax.ShapeDtypeStruct((_B,), jnp.float32)),
        mesh=mesh,
        scratch_types=[
            pltpu.VMEM((_BPW,), jnp.int32),
            pltpu.VMEM((_BPW,), jnp.int32),
            pltpu.VMEM((_BPW,), jnp.int32),
            pltpu.VMEM((_BPW,), jnp.int32),
            pltpu.VMEM((_BPW,), jnp.float32),
        ] + set_scratch,
        compiler_params=pltpu.CompilerParams(needs_layout_passes=False,
                                             use_tc_tiling_on_sc=True),
    )
    return f(*idxs, ent3, rel2)


# speedup vs baseline: 1.0021x; 1.0021x over previous
import functools

import jax
import jax.numpy as jnp
from jax import lax
from jax.experimental import pallas as pl
from jax.experimental.pallas import tpu as pltpu
from jax.experimental.pallas import tpu_sc as plsc

_B = 16384
_D = 64
_NC = 2
_NS = 16
_NW = _NC * _NS
_BPW = _B // _NW
_G = 16
_NG = _BPW // _G
_NB = 3


def _body(ph, pr, pt, nh, nr, nt, ent3, rel2, pos_out, neg_out,
          rh, rr, rt, dr, out_v, *setargs):
    wid = lax.axis_index("s") * _NC + lax.axis_index("c")
    sets = [(setargs[3 * b], setargs[3 * b + 1], setargs[3 * b + 2])
            for b in range(_NB)]

    def enqueues(g, b):
        bht, brl, sem = sets[b]
        v_h = rh[pl.ds(g * _G, _G)]
        v_t = rt[pl.ds(g * _G, _G)]
        thunks = [lambda: pltpu.async_copy(
            rel2.at[dr.at[pl.ds(g * _G, _G)]], brl, sem)]
        for i in range(_G):
            thunks.append(functools.partial(
                lambda i: pltpu.async_copy(
                    ent3.at[lax.shift_right_logical(v_h[i], 3)],
                    bht.at[i], sem), i))
            thunks.append(functools.partial(
                lambda i: pltpu.async_copy(
                    ent3.at[lax.shift_right_logical(v_t[i], 3)],
                    bht.at[i + _G], sem), i))
        return thunks

    def fire(g, b):
        for th in enqueues(g, b):
            th()

    def drain(b):
        bht, brl, sem = sets[b]
        pltpu.make_async_copy(ent3.at[pl.ds(0, 2 * _G)], bht, sem).wait()
        pltpu.make_async_copy(rel2.at[pl.ds(0, _G)], brl, sem).wait()

    def compute(g, b, fire_thunks=()):
        bht, brl, _ = sets[b]
        v_h = rh[pl.ds(g * _G, _G)]
        v_t = rt[pl.ds(g * _G, _G)]
        rows = lax.iota(jnp.int32, 16)
        rows_t = rows + _G
        k_h = v_h & 7
        k_t = v_t & 7
        base_r = (rr[pl.ds(g * _G, _G)] & 1) * _D
        acc = jnp.zeros((16,), jnp.float32)
        thunks = list(fire_thunks)
        for j in range(_D):
            jv = lax.broadcast(j, (16,))
            h = plsc.load_gather(bht, [rows, k_h, jv])
            t = plsc.load_gather(bht, [rows_t, k_t, jv])
            r = plsc.load_gather(brl, [rows, base_r + j])
            d = h + r - t
            acc = acc + d * d
            if thunks and j % 2 == 0:
                thunks.pop()()
        for th in thunks:
            th()
        out_v[pl.ds(g * _G, 16)] = acc

    def compute_tail(g, b):
        bht, brl, _ = sets[b]
        v_h = rh[pl.ds(g * _G, _G)]
        v_t = rt[pl.ds(g * _G, _G)]
        rows = lax.iota(jnp.int32, 16)
        rows_t = rows + _G
        k_h = v_h & 7
        k_t = v_t & 7
        base_r = (rr[pl.ds(g * _G, _G)] & 1) * _D

        def jbody(j4, acc):
            for u in range(4):
                j = j4 * 4 + u
                jv = lax.broadcast(j, (16,))
                h = plsc.load_gather(bht, [rows, k_h, jv])
                t = plsc.load_gather(bht, [rows_t, k_t, jv])
                r = plsc.load_gather(brl, [rows, base_r + j])
                d = h + r - t
                acc = acc + d * d
            return acc

        acc = lax.fori_loop(0, _D // 4, jbody, jnp.zeros((16,), jnp.float32))
        out_v[pl.ds(g * _G, 16)] = acc

    def do_term(hi, ri, ti, out_hbm):
        pltpu.sync_copy(hi.at[wid], rh)
        pltpu.sync_copy(ri.at[wid], rr)
        pltpu.sync_copy(ti.at[wid], rt)
        for s in range(_BPW // 16):
            v = rr[pl.ds(s * 16, 16)]
            dr[pl.ds(s * 16, 16)] = lax.shift_right_logical(v, 1)
        fire(0, 0)
        fire(1, 1)

        def qbody(q, _):
            for b in range(_NB):
                g = q * _NB + b
                drain(b)
                compute(g, b, enqueues(g + 2, (b + 2) % _NB))
            return 0

        lax.fori_loop(0, (_NG - 2) // _NB, qbody, 0)
        for g in range(_NG - 2, _NG):
            b = g % _NB
            drain(b)
            compute_tail(g, b)
        pltpu.sync_copy(out_v, out_hbm.at[pl.ds(wid * _BPW, _BPW)])

    do_term(ph, pr, pt, pos_out)
    do_term(nh, nr, nt, neg_out)


@functools.partial(jax.jit)
def kernel(ph, pr, pt, nh, nr, nt, ent_embed, rel_embed):
    idxs = [x.astype(jnp.int32).reshape(_NW, _BPW)
            for x in (ph, pr, pt, nh, nr, nt)]
    ent3 = ent_embed.reshape(ent_embed.shape[0] // 8, 8, _D)
    rel2 = rel_embed.reshape(rel_embed.shape[0] // 2, 2 * _D)
    mesh = plsc.VectorSubcoreMesh(core_axis_name="c", subcore_axis_name="s",
                                  num_cores=_NC, num_subcores=_NS)
    set_scratch = []
    for _ in range(_NB):
        set_scratch += [
            pltpu.VMEM((2 * _G, 8, _D), jnp.float32),
            pltpu.VMEM((_G, 2 * _D), jnp.float32),
            pltpu.SemaphoreType.DMA,
        ]
    f = pl.kernel(
        _body,
        out_type=(jax.ShapeDtypeStruct((_B,), jnp.float32),
                  jax.ShapeDtypeStruct((_B,), jnp.float32)),
        mesh=mesh,
        scratch_types=[
            pltpu.VMEM((_BPW,), jnp.int32),
            pltpu.VMEM((_BPW,), jnp.int32),
            pltpu.VMEM((_BPW,), jnp.int32),
            pltpu.VMEM((_BPW,), jnp.int32),
            pltpu.VMEM((_BPW,), jnp.float32),
        ] + set_scratch,
        compiler_params=pltpu.CompilerParams(needs_layout_passes=False,
                                             use_tc_tiling_on_sc=True),
    )
    return f(*idxs, ent3, rel2)


# speedup vs baseline: 1.0232x; 1.0211x over previous
import functools

import jax
import jax.numpy as jnp
from jax import lax
from jax.experimental import pallas as pl
from jax.experimental.pallas import tpu as pltpu
from jax.experimental.pallas import tpu_sc as plsc

_B = 16384
_D = 64
_NC = 2
_NS = 16
_NW = _NC * _NS
_BPW = _B // _NW
_G = 16
_NG = _BPW // _G
_NB = 3


def _body(ph, pr, pt, nh, nr, nt, ent3, rel2, pos_out, neg_out,
          rh, rr, rt, dr, out_v, *setargs):
    wid = lax.axis_index("s") * _NC + lax.axis_index("c")
    sets = [(setargs[3 * b], setargs[3 * b + 1], setargs[3 * b + 2])
            for b in range(_NB)]

    def enqueues(g, b):
        bht, brl, sem = sets[b]
        v_h = rh[pl.ds(g * _G, _G)]
        v_t = rt[pl.ds(g * _G, _G)]
        thunks = [lambda: pltpu.async_copy(
            rel2.at[dr.at[pl.ds(g * _G, _G)]], brl, sem)]
        for i in range(_G):
            thunks.append(functools.partial(
                lambda i: pltpu.async_copy(
                    ent3.at[lax.shift_right_logical(v_h[i], 3)],
                    bht.at[i], sem), i))
            thunks.append(functools.partial(
                lambda i: pltpu.async_copy(
                    ent3.at[lax.shift_right_logical(v_t[i], 3)],
                    bht.at[i + _G], sem), i))
        return thunks

    def fire(g, b):
        for th in enqueues(g, b):
            th()

    def drain(b):
        bht, brl, sem = sets[b]
        pltpu.make_async_copy(ent3.at[pl.ds(0, 2 * _G)], bht, sem).wait()
        pltpu.make_async_copy(rel2.at[pl.ds(0, _G)], brl, sem).wait()

    def compute_tail(g, b):
        bht, brl, _ = sets[b]
        v_h = rh[pl.ds(g * _G, _G)]
        v_t = rt[pl.ds(g * _G, _G)]
        rows = lax.iota(jnp.int32, 16)
        rows_t = rows + _G
        k_h = v_h & 7
        k_t = v_t & 7
        base_r = (rr[pl.ds(g * _G, _G)] & 1) * _D

        def jbody(j4, acc):
            for u in range(4):
                j = j4 * 4 + u
                jv = lax.broadcast(j, (16,))
                h = plsc.load_gather(bht, [rows, k_h, jv])
                t = plsc.load_gather(bht, [rows_t, k_t, jv])
                r = plsc.load_gather(brl, [rows, base_r + j])
                d = h + r - t
                acc = acc + d * d
            return acc

        acc = lax.fori_loop(0, _D // 4, jbody, jnp.zeros((16,), jnp.float32))
        out_v[pl.ds(g * _G, 16)] = acc

    def do_term(hi, ri, ti, out_hbm):
        pltpu.sync_copy(hi.at[wid], rh)
        pltpu.sync_copy(ri.at[wid], rr)
        pltpu.sync_copy(ti.at[wid], rt)
        for s in range(_BPW // 16):
            v = rr[pl.ds(s * 16, 16)]
            dr[pl.ds(s * 16, 16)] = lax.shift_right_logical(v, 1)
        for b in range(_NB):
            fire(b, b)

        def qbody(q, _):
            for b in range(_NB):
                g = q * _NB + b
                drain(b)
                compute_tail(g, b)

                @pl.when(g + _NB < _NG)
                def _():
                    fire(g + _NB, b)
            return 0

        lax.fori_loop(0, 10, qbody, 0)
        for g in range(30, _NG):
            b = g % _NB
            drain(b)
            compute_tail(g, b)
        pltpu.sync_copy(out_v, out_hbm.at[pl.ds(wid * _BPW, _BPW)])

    do_term(ph, pr, pt, pos_out)
    do_term(nh, nr, nt, neg_out)


@functools.partial(jax.jit)
def kernel(ph, pr, pt, nh, nr, nt, ent_embed, rel_embed):
    idxs = [x.astype(jnp.int32).reshape(_NW, _BPW)
            for x in (ph, pr, pt, nh, nr, nt)]
    ent3 = ent_embed.reshape(ent_embed.shape[0] // 8, 8, _D)
    rel2 = rel_embed.reshape(rel_embed.shape[0] // 2, 2 * _D)
    mesh = plsc.VectorSubcoreMesh(core_axis_name="c", subcore_axis_name="s",
                                  num_cores=_NC, num_subcores=_NS)
    set_scratch = []
    for _ in range(_NB):
        set_scratch += [
            pltpu.VMEM((2 * _G, 8, _D), jnp.float32),
            pltpu.VMEM((_G, 2 * _D), jnp.float32),
            pltpu.SemaphoreType.DMA,
        ]
    f = pl.kernel(
        _body,
        out_type=(jax.ShapeDtypeStruct((_B,), jnp.float32),
                  jax.ShapeDtypeStruct((_B,), jnp.float32)),
        mesh=mesh,
        scratch_types=[
            pltpu.VMEM((_BPW,), jnp.int32),
            pltpu.VMEM((_BPW,), jnp.int32),
            pltpu.VMEM((_BPW,), jnp.int32),
            pltpu.VMEM((_BPW,), jnp.int32),
            pltpu.VMEM((_BPW,), jnp.float32),
        ] + set_scratch,
        compiler_params=pltpu.CompilerParams(needs_layout_passes=False,
                                             use_tc_tiling_on_sc=True),
    )
    return f(*idxs, ent3, rel2)


# speedup vs baseline: 1.0328x; 1.0094x over previous
import functools

import jax
import jax.numpy as jnp
from jax import lax
from jax.experimental import pallas as pl
from jax.experimental.pallas import tpu as pltpu
from jax.experimental.pallas import tpu_sc as plsc

_B = 16384
_D = 64
_NC = 2
_NS = 16
_NW = _NC * _NS
_BPW = _B // _NW
_G = 16
_NG = _BPW // _G
_NB = 3


def _body(ph, pr, pt, nh, nr, nt, ent3, rel2, pos_out, neg_out,
          rh, rr, rt, dr, out_v, *setargs):
    wid = lax.axis_index("s") * _NC + lax.axis_index("c")
    sets = [(setargs[3 * b], setargs[3 * b + 1], setargs[3 * b + 2])
            for b in range(_NB)]

    def enqueues(g, b):
        bht, brl, sem = sets[b]
        v_h = rh[pl.ds(g * _G, _G)]
        v_t = rt[pl.ds(g * _G, _G)]
        thunks = [lambda: pltpu.async_copy(
            rel2.at[dr.at[pl.ds(g * _G, _G)]], brl, sem)]
        for i in range(_G):
            thunks.append(functools.partial(
                lambda i: pltpu.async_copy(
                    ent3.at[lax.shift_right_logical(v_h[i], 3)],
                    bht.at[i], sem), i))
            thunks.append(functools.partial(
                lambda i: pltpu.async_copy(
                    ent3.at[lax.shift_right_logical(v_t[i], 3)],
                    bht.at[i + _G], sem), i))
        return thunks

    def fire(g, b):
        for th in enqueues(g, b):
            th()

    def drain(b):
        bht, brl, sem = sets[b]
        pltpu.make_async_copy(ent3.at[pl.ds(0, 2 * _G)], bht, sem).wait()
        pltpu.make_async_copy(rel2.at[pl.ds(0, _G)], brl, sem).wait()

    def compute_tail(g, b):
        bht, brl, _ = sets[b]
        v_h = rh[pl.ds(g * _G, _G)]
        v_t = rt[pl.ds(g * _G, _G)]
        rows = lax.iota(jnp.int32, 16)
        rows_t = rows + _G
        k_h = v_h & 7
        k_t = v_t & 7
        base_r = (rr[pl.ds(g * _G, _G)] & 1) * _D

        def jbody(j8, acc):
            for u in range(8):
                j = j8 * 8 + u
                jv = lax.broadcast(j, (16,))
                h = plsc.load_gather(bht, [rows, k_h, jv])
                t = plsc.load_gather(bht, [rows_t, k_t, jv])
                r = plsc.load_gather(brl, [rows, base_r + j])
                d = h + r - t
                acc = acc + d * d
            return acc

        acc = lax.fori_loop(0, _D // 8, jbody, jnp.zeros((16,), jnp.float32))
        out_v[pl.ds(g * _G, 16)] = acc

    def do_term(hi, ri, ti, out_hbm):
        pltpu.sync_copy(hi.at[wid], rh)
        pltpu.sync_copy(ri.at[wid], rr)
        pltpu.sync_copy(ti.at[wid], rt)
        for s in range(_BPW // 16):
            v = rr[pl.ds(s * 16, 16)]
            dr[pl.ds(s * 16, 16)] = lax.shift_right_logical(v, 1)
        for b in range(_NB):
            fire(b, b)

        def qbody(q, _):
            for b in range(_NB):
                g = q * _NB + b
                drain(b)
                compute_tail(g, b)

                @pl.when(g + _NB < _NG)
                def _():
                    fire(g + _NB, b)
            return 0

        lax.fori_loop(0, 10, qbody, 0)
        for g in range(30, _NG):
            b = g % _NB
            drain(b)
            compute_tail(g, b)
        pltpu.sync_copy(out_v, out_hbm.at[pl.ds(wid * _BPW, _BPW)])

    do_term(ph, pr, pt, pos_out)
    do_term(nh, nr, nt, neg_out)


@functools.partial(jax.jit)
def kernel(ph, pr, pt, nh, nr, nt, ent_embed, rel_embed):
    idxs = [x.astype(jnp.int32).reshape(_NW, _BPW)
            for x in (ph, pr, pt, nh, nr, nt)]
    ent3 = ent_embed.reshape(ent_embed.shape[0] // 8, 8, _D)
    rel2 = rel_embed.reshape(rel_embed.shape[0] // 2, 2 * _D)
    mesh = plsc.VectorSubcoreMesh(core_axis_name="c", subcore_axis_name="s",
                                  num_cores=_NC, num_subcores=_NS)
    set_scratch = []
    for _ in range(_NB):
        set_scratch += [
            pltpu.VMEM((2 * _G, 8, _D), jnp.float32),
            pltpu.VMEM((_G, 2 * _D), jnp.float32),
            pltpu.SemaphoreType.DMA,
        ]
    f = pl.kernel(
        _body,
        out_type=(jax.ShapeDtypeStruct((_B,), jnp.float32),
                  jax.ShapeDtypeStruct((_B,), jnp.float32)),
        mesh=mesh,
        scratch_types=[
            pltpu.VMEM((_BPW,), jnp.int32),
            pltpu.VMEM((_BPW,), jnp.int32),
            pltpu.VMEM((_BPW,), jnp.int32),
            pltpu.VMEM((_BPW,), jnp.int32),
            pltpu.VMEM((_BPW,), jnp.float32),
        ] + set_scratch,
        compiler_params=pltpu.CompilerParams(needs_layout_passes=False,
                                             use_tc_tiling_on_sc=True),
    )
    return f(*idxs, ent3, rel2)


# speedup vs baseline: 1.0330x; 1.0001x over previous
import functools

import jax
import jax.numpy as jnp
from jax import lax
from jax.experimental import pallas as pl
from jax.experimental.pallas import tpu as pltpu
from jax.experimental.pallas import tpu_sc as plsc

_B = 16384
_D = 64
_NC = 2
_NS = 16
_NW = _NC * _NS
_BPW = _B // _NW
_G = 16
_NG = _BPW // _G
_NB = 3


def _body(ph, pr, pt, nh, nr, nt, ent3, rel2, pos_out, neg_out,
          rh, rr, rt, dr, out_v, *setargs):
    wid = lax.axis_index("s") * _NC + lax.axis_index("c")
    sets = [(setargs[3 * b], setargs[3 * b + 1], setargs[3 * b + 2])
            for b in range(_NB)]

    def enqueues(g, b):
        bht, brl, sem = sets[b]
        v_h = rh[pl.ds(g * _G, _G)]
        v_t = rt[pl.ds(g * _G, _G)]
        thunks = [lambda: pltpu.async_copy(
            rel2.at[dr.at[pl.ds(g * _G, _G)]], brl, sem)]
        for i in range(_G):
            thunks.append(functools.partial(
                lambda i: pltpu.async_copy(
                    ent3.at[lax.shift_right_logical(v_h[i], 3)],
                    bht.at[i], sem), i))
            thunks.append(functools.partial(
                lambda i: pltpu.async_copy(
                    ent3.at[lax.shift_right_logical(v_t[i], 3)],
                    bht.at[i + _G], sem), i))
        return thunks

    def fire(g, b):
        for th in enqueues(g, b):
            th()

    def drain(b):
        bht, brl, sem = sets[b]
        pltpu.make_async_copy(ent3.at[pl.ds(0, 2 * _G)], bht, sem).wait()
        pltpu.make_async_copy(rel2.at[pl.ds(0, _G)], brl, sem).wait()

    def compute_tail(g, b):
        bht, brl, _ = sets[b]
        v_h = rh[pl.ds(g * _G, _G)]
        v_t = rt[pl.ds(g * _G, _G)]
        rows = lax.iota(jnp.int32, 16)
        rows_t = rows + _G
        k_h = v_h & 7
        k_t = v_t & 7
        base_r = (rr[pl.ds(g * _G, _G)] & 1) * _D

        @plsc.parallel_loop(0, _D, unroll=8,
                            carry=jnp.zeros((16,), jnp.float32))
        def acc(j, acc):
            jv = lax.broadcast(j, (16,))
            h = plsc.load_gather(bht, [rows, k_h, jv])
            t = plsc.load_gather(bht, [rows_t, k_t, jv])
            r = plsc.load_gather(brl, [rows, base_r + j])
            d = h + r - t
            return acc + d * d
        out_v[pl.ds(g * _G, 16)] = acc

    def do_term(hi, ri, ti, out_hbm):
        pltpu.sync_copy(hi.at[wid], rh)
        pltpu.sync_copy(ri.at[wid], rr)
        pltpu.sync_copy(ti.at[wid], rt)
        for s in range(_BPW // 16):
            v = rr[pl.ds(s * 16, 16)]
            dr[pl.ds(s * 16, 16)] = lax.shift_right_logical(v, 1)
        for b in range(_NB):
            fire(b, b)

        def qbody(q, _):
            for b in range(_NB):
                g = q * _NB + b
                drain(b)
                compute_tail(g, b)

                @pl.when(g + _NB < _NG)
                def _():
                    fire(g + _NB, b)
            return 0

        lax.fori_loop(0, 10, qbody, 0)
        for g in range(30, _NG):
            b = g % _NB
            drain(b)
            compute_tail(g, b)
        pltpu.sync_copy(out_v, out_hbm.at[pl.ds(wid * _BPW, _BPW)])

    do_term(ph, pr, pt, pos_out)
    do_term(nh, nr, nt, neg_out)


@functools.partial(jax.jit)
def kernel(ph, pr, pt, nh, nr, nt, ent_embed, rel_embed):
    idxs = [x.astype(jnp.int32).reshape(_NW, _BPW)
            for x in (ph, pr, pt, nh, nr, nt)]
    ent3 = ent_embed.reshape(ent_embed.shape[0] // 8, 8, _D)
    rel2 = rel_embed.reshape(rel_embed.shape[0] // 2, 2 * _D)
    mesh = plsc.VectorSubcoreMesh(core_axis_name="c", subcore_axis_name="s",
                                  num_cores=_NC, num_subcores=_NS)
    set_scratch = []
    for _ in range(_NB):
        set_scratch += [
            pltpu.VMEM((2 * _G, 8, _D), jnp.float32),
            pltpu.VMEM((_G, 2 * _D), jnp.float32),
            pltpu.SemaphoreType.DMA,
        ]
    f = pl.kernel(
        _body,
        out_type=(jax.ShapeDtypeStruct((_B,), jnp.float32),
                  jax.ShapeDtypeStruct((_B,), jnp.float32)),
        mesh=mesh,
        scratch_types=[
            pltpu.VMEM((_BPW,), jnp.int32),
            pltpu.VMEM((_BPW,), jnp.int32),
            pltpu.VMEM((_BPW,), jnp.int32),
            pltpu.VMEM((_BPW,), jnp.int32),
            pltpu.VMEM((_BPW,), jnp.float32),
        ] + set_scratch,
        compiler_params=pltpu.CompilerParams(needs_layout_passes=False,
                                             use_tc_tiling_on_sc=True),
    )
    return f(*idxs, ent3, rel2)


# speedup vs baseline: 1.0338x; 1.0008x over previous
import functools

import jax
import jax.numpy as jnp
from jax import lax
from jax.experimental import pallas as pl
from jax.experimental.pallas import tpu as pltpu
from jax.experimental.pallas import tpu_sc as plsc

_B = 16384
_D = 64
_NC = 2
_NS = 16
_NW = _NC * _NS
_BPW = _B // _NW
_G = 16
_NG = _BPW // _G
_NB = 3


def _body(ph, pr, pt, nh, nr, nt, ent3, rel2, pos_out, neg_out,
          rh, rr, rt, dr, out_v, *setargs):
    wid = lax.axis_index("s") * _NC + lax.axis_index("c")
    sets = [(setargs[3 * b], setargs[3 * b + 1], setargs[3 * b + 2])
            for b in range(_NB)]

    def fire(g, b):
        bht, brl, sem = sets[b]
        v_h = rh[pl.ds(g * _G, _G)]
        v_t = rt[pl.ds(g * _G, _G)]
        pltpu.async_copy(rel2.at[dr.at[pl.ds(g * _G, _G)]], brl, sem)
        for i in range(_G):
            pltpu.async_copy(ent3.at[lax.shift_right_logical(v_h[i], 3)],
                             bht.at[i], sem)
            pltpu.async_copy(ent3.at[lax.shift_right_logical(v_t[i], 3)],
                             bht.at[i + _G], sem)

    def drain(b):
        bht, brl, sem = sets[b]
        pltpu.make_async_copy(ent3.at[pl.ds(0, 2 * _G)], bht, sem).wait()
        pltpu.make_async_copy(rel2.at[pl.ds(0, _G)], brl, sem).wait()

    def compute(g, b):
        bht, brl, _ = sets[b]
        v_h = rh[pl.ds(g * _G, _G)]
        v_t = rt[pl.ds(g * _G, _G)]
        rows = lax.iota(jnp.int32, 16)
        rows_t = rows + _G
        k_h = v_h & 7
        k_t = v_t & 7
        base_r = (rr[pl.ds(g * _G, _G)] & 1) * _D

        def jbody(j8, acc):
            for u in range(8):
                j = j8 * 8 + u
                jv = lax.broadcast(j, (16,))
                h = plsc.load_gather(bht, [rows, k_h, jv])
                t = plsc.load_gather(bht, [rows_t, k_t, jv])
                r = plsc.load_gather(brl, [rows, base_r + j])
                d = h + r - t
                acc = acc + d * d
            return acc

        acc = lax.fori_loop(0, _D // 8, jbody, jnp.zeros((16,), jnp.float32))
        out_v[pl.ds(g * _G, 16)] = acc

    def do_term(hi, ri, ti, out_hbm):
        pltpu.sync_copy(hi.at[wid], rh)
        pltpu.sync_copy(ri.at[wid], rr)
        pltpu.sync_copy(ti.at[wid], rt)
        for s in range(_BPW // 16):
            v = rr[pl.ds(s * 16, 16)]
            dr[pl.ds(s * 16, 16)] = lax.shift_right_logical(v, 1)
        for b in range(_NB):
            fire(b, b)

        def qbody(q, _):
            for b in range(_NB):
                g = q * _NB + b
                drain(b)
                compute(g, b)

                @pl.when(g + _NB < _NG)
                def _():
                    fire(g + _NB, b)
            return 0

        lax.fori_loop(0, 10, qbody, 0)
        for g in range(30, _NG):
            b = g % _NB
            drain(b)
            compute(g, b)
        pltpu.sync_copy(out_v, out_hbm.at[pl.ds(wid * _BPW, _BPW)])

    do_term(ph, pr, pt, pos_out)
    do_term(nh, nr, nt, neg_out)


@functools.partial(jax.jit)
def kernel(ph, pr, pt, nh, nr, nt, ent_embed, rel_embed):
    idxs = [x.astype(jnp.int32).reshape(_NW, _BPW)
            for x in (ph, pr, pt, nh, nr, nt)]
    ent3 = ent_embed.reshape(ent_embed.shape[0] // 8, 8, _D)
    rel2 = rel_embed.reshape(rel_embed.shape[0] // 2, 2 * _D)
    mesh = plsc.VectorSubcoreMesh(core_axis_name="c", subcore_axis_name="s",
                                  num_cores=_NC, num_subcores=_NS)
    set_scratch = []
    for _ in range(_NB):
        set_scratch += [
            pltpu.VMEM((2 * _G, 8, _D), jnp.float32),
            pltpu.VMEM((_G, 2 * _D), jnp.float32),
            pltpu.SemaphoreType.DMA,
        ]
    f = pl.kernel(
        _body,
        out_type=(jax.ShapeDtypeStruct((_B,), jnp.float32),
                  jax.ShapeDtypeStruct((_B,), jnp.float32)),
        mesh=mesh,
        scratch_types=[
            pltpu.VMEM((_BPW,), jnp.int32),
            pltpu.VMEM((_BPW,), jnp.int32),
            pltpu.VMEM((_BPW,), jnp.int32),
            pltpu.VMEM((_BPW,), jnp.int32),
            pltpu.VMEM((_BPW,), jnp.float32),
        ] + set_scratch,
        compiler_params=pltpu.CompilerParams(needs_layout_passes=False,
                                             use_tc_tiling_on_sc=True),
    )
    return f(*idxs, ent3, rel2)
